# Initial kernel scaffold; baseline (speedup 1.0000x reference)
import functools

import jax
import jax.numpy as jnp
from jax import lax
from jax.experimental import pallas as pl
from jax.experimental.pallas import tpu as pltpu
from jax.experimental.pallas import tpu_sc as plsc

N = 10000
E = 320000
D = 128
NP = 10240
NW = 32
EPT = NP
CH = EPT // 128
ROWS_PT = NP // 16
G = 128
R = 1024

f32 = jnp.float32
i32 = jnp.int32

_mesh = plsc.VectorSubcoreMesh(
    core_axis_name="c", subcore_axis_name="s", num_cores=2, num_subcores=16)


@functools.partial(
    pl.kernel,
    out_type=jax.ShapeDtypeStruct((2, 2, NP), f32),
    mesh=_mesh,
    scratch_types=[
        pltpu.VMEM((CH, 128), i32),
        pltpu.VMEM((CH, 128), i32),
        pltpu.VMEM((128,), f32),
        pltpu.VMEM((ROWS_PT,), f32),
        pltpu.VMEM_SHARED((NP,), f32),
        pltpu.VMEM_SHARED((NP,), f32),
        pltpu.SemaphoreType.DMA,
    ])
def _deg_kernel(src_hbm, dst_hbm, z1_hbm, out_hbm,
                src_v, dst_v, ones_v, zb_v, oacc, iacc, sem):
    c = lax.axis_index("c")
    s = lax.axis_index("s")
    wid = s * 2 + c
    for i in range(8):
        ones_v[pl.ds(16 * i, 16)] = jnp.ones((16,), f32)
    pltpu.sync_copy(z1_hbm, zb_v)
    pltpu.sync_copy(zb_v, oacc.at[pl.ds(s * ROWS_PT, ROWS_PT)])
    pltpu.sync_copy(zb_v, iacc.at[pl.ds(s * ROWS_PT, ROWS_PT)])
    pltpu.sync_copy(src_hbm.at[wid], src_v)
    pltpu.sync_copy(dst_hbm.at[wid], dst_v)
    plsc.subcore_barrier()

    def body(j, carry):
        pltpu.sync_copy(ones_v, oacc.at[src_v.at[j]], add=True)
        pltpu.sync_copy(ones_v, iacc.at[dst_v.at[j]], add=True)
        return carry

    lax.fori_loop(0, CH, body, 0)
    plsc.subcore_barrier()
    sl = pl.ds(s * ROWS_PT, ROWS_PT)
    pltpu.sync_copy(oacc.at[sl], out_hbm.at[c, 0, sl])
    pltpu.sync_copy(iacc.at[sl], out_hbm.at[c, 1, sl])


@functools.partial(
    pl.kernel,
    out_type=jax.ShapeDtypeStruct((2, NP, D), f32),
    mesh=_mesh,
    scratch_types=[
        pltpu.VMEM((CH, 128), i32),
        pltpu.VMEM((CH, 128), i32),
        pltpu.VMEM((128, D), f32),
        pltpu.VMEM_SHARED((NP, D), f32),
        pltpu.SemaphoreType.DMA,
    ])
def _hop_kernel(src_hbm, dst_hbm, hs_hbm, z2_hbm, out_hbm,
                src_v, dst_v, rows_v, acc, sem):
    c = lax.axis_index("c")
    s = lax.axis_index("s")
    wid = s * 2 + c
    pltpu.sync_copy(z2_hbm, rows_v)
    for i in range(ROWS_PT // 128):
        pltpu.sync_copy(rows_v, acc.at[pl.ds(s * ROWS_PT + i * 128, 128)])
    pltpu.sync_copy(src_hbm.at[wid], src_v)
    pltpu.sync_copy(dst_hbm.at[wid], dst_v)
    plsc.subcore_barrier()

    def body(j, carry):
        pltpu.async_copy(hs_hbm.at[src_v.at[j]], rows_v, sem).wait()
        pltpu.sync_copy(rows_v, acc.at[dst_v.at[j]], add=True)
        return carry

    lax.fori_loop(0, CH, body, 0)
    plsc.subcore_barrier()
    for i in range(ROWS_PT // 128):
        sl = pl.ds(s * ROWS_PT + i * 128, 128)
        pltpu.sync_copy(acc.at[sl], out_hbm.at[c, sl])


def _norm_body(degt_ref, feat_ref, norm2_ref, hs_ref):
    dg = degt_ref[...]
    od = dg[:, 0:1] + dg[:, 2:3]
    idg = dg[:, 1:2] + dg[:, 3:4]
    ns = 1.0 / jnp.sqrt(jnp.maximum(od, 1.0))
    nd = 1.0 / jnp.sqrt(jnp.maximum(idg, 1.0))
    norm2_ref[...] = jnp.concatenate([ns, nd], axis=1)
    hs_ref[...] = feat_ref[...] * ns


def _upd1_body(aggp_ref, norm2_ref, feat_ref, hs_ref):
    a = aggp_ref[0] + aggp_ref[1]
    n2 = norm2_ref[...]
    h = 0.5 * (a * n2[:, 1:2]) + 0.5 * feat_ref[...]
    hs_ref[...] = h * n2[:, 0:1]


def _upd2_body(aggp_ref, norm2_ref, feat_ref, t_ref, l_ref, h_ref, s_ref):
    r = pl.program_id(0)
    a = aggp_ref[0] + aggp_ref[1]
    n2 = norm2_ref[...]
    h = 0.5 * (a * n2[:, 1:2]) + 0.5 * feat_ref[...]
    h_ref[...] = h
    gid = t_ref[...] * 16 + l_ref[...]
    iota = lax.broadcasted_iota(i32, (R, G), 1)
    onehot = (iota == gid).astype(f32)
    onecol = (iota == 0).astype(f32)
    y = jnp.concatenate([h, h * h, onecol], axis=1)

    @pl.when(r == 0)
    def _init():
        s_ref[...] = jnp.zeros_like(s_ref)

    s_ref[...] += lax.dot_general(
        onehot, y, (((0,), (0,)), ((), ())),
        preferred_element_type=f32, precision=lax.Precision.HIGHEST)


def _final_body(h_ref, t_ref, l_ref, s_ref, w1t_ref, b1_ref, w2t_ref, b2_ref,
                o_ref):
    s = s_ref[...]
    cnt = jnp.maximum(s[:, 2 * G:2 * G + 1], 1.0)
    mean = s[:, 0:G] / cnt
    ex2 = s[:, G:2 * G] / cnt
    var = jnp.maximum(ex2 - mean * mean, 0.0)
    inv = 1.0 / jnp.sqrt(var + 1e-5)
    gid = t_ref[...] * 16 + l_ref[...]
    iota = lax.broadcasted_iota(i32, (R, G), 1)
    onehot = (iota == gid).astype(f32)
    hp = lax.Precision.HIGHEST
    mean_r = lax.dot_general(onehot, mean, (((1,), (0,)), ((), ())),
                             preferred_element_type=f32, precision=hp)
    inv_r = lax.dot_general(onehot, inv, (((1,), (0,)), ((), ())),
                            preferred_element_type=f32, precision=hp)
    hn = (h_ref[...] - mean_r) * inv_r
    z = lax.dot_general(hn, w1t_ref[...], (((1,), (0,)), ((), ())),
                        preferred_element_type=f32, precision=hp)
    z = jnp.maximum(z + b1_ref[...], 0.0)
    o = lax.dot_general(z, w2t_ref[...], (((1,), (0,)), ((), ())),
                        preferred_element_type=f32, precision=hp)
    o_ref[...] = jax.nn.sigmoid(o + b2_ref[...])


def _row_spec(width):
    return pl.BlockSpec((R, width), lambda r: (r, 0))


def _full_spec(shape):
    return pl.BlockSpec(shape, lambda r: tuple(0 for _ in shape))


def kernel(features, edge_index, labels, times, W1, b1, W2, b2):
    src = edge_index[0]
    dst = edge_index[1]
    npad = NP - N
    epad = NW * EPT - E
    pad_idx = (N + (jnp.arange(epad, dtype=i32) % npad)).astype(i32)
    src_p = jnp.concatenate([src, pad_idx]).reshape(NW, CH, 128)
    dst_p = jnp.concatenate([dst, pad_idx]).reshape(NW, CH, 128)
    feat_p = jnp.pad(features, ((0, npad), (0, 0)))
    t_p = jnp.pad(times, (0, npad), constant_values=8).reshape(NP, 1)
    l_p = jnp.pad(labels, (0, npad)).reshape(NP, 1)
    z1 = jnp.zeros((ROWS_PT,), f32)
    z2 = jnp.zeros((128, D), f32)

    degp = _deg_kernel(src_p, dst_p, z1)
    degt = jnp.transpose(degp.reshape(4, NP))

    grid = (NP // R,)
    norm2, hs0 = pl.pallas_call(
        _norm_body,
        grid=grid,
        in_specs=[_row_spec(4), _row_spec(D)],
        out_specs=[_row_spec(2), _row_spec(D)],
        out_shape=[jax.ShapeDtypeStruct((NP, 2), f32),
                   jax.ShapeDtypeStruct((NP, D), f32)],
    )(degt, feat_p)

    aggp1 = _hop_kernel(src_p, dst_p, hs0, z2)
    hs1 = pl.pallas_call(
        _upd1_body,
        grid=grid,
        in_specs=[pl.BlockSpec((2, R, D), lambda r: (0, r, 0)),
                  _row_spec(2), _row_spec(D)],
        out_specs=_row_spec(D),
        out_shape=jax.ShapeDtypeStruct((NP, D), f32),
    )(aggp1, norm2, feat_p)

    aggp2 = _hop_kernel(src_p, dst_p, hs1, z2)
    h2, stats = pl.pallas_call(
        _upd2_body,
        grid=grid,
        in_specs=[pl.BlockSpec((2, R, D), lambda r: (0, r, 0)),
                  _row_spec(2), _row_spec(D), _row_spec(1), _row_spec(1)],
        out_specs=[_row_spec(D), _full_spec((G, 3 * G))],
        out_shape=[jax.ShapeDtypeStruct((NP, D), f32),
                   jax.ShapeDtypeStruct((G, 3 * G), f32)],
    )(aggp2, norm2, feat_p, t_p, l_p)

    out = pl.pallas_call(
        _final_body,
        grid=grid,
        in_specs=[_row_spec(D), _row_spec(1), _row_spec(1),
                  _full_spec((G, 3 * G)), _full_spec((D, D)),
                  _full_spec((1, D)), _full_spec((D, 32)),
                  _full_spec((1, 32))],
        out_specs=_row_spec(32),
        out_shape=jax.ShapeDtypeStruct((NP, 32), f32),
    )(h2, t_p, l_p, stats, W1.T, b1.reshape(1, D), W2.T, b2.reshape(1, 32))
    return out[:N]


# speedup vs baseline: 7.8218x; 7.8218x over previous
import functools

import jax
import jax.numpy as jnp
from jax import lax
from jax.experimental import pallas as pl
from jax.experimental.pallas import tpu as pltpu
from jax.experimental.pallas import tpu_sc as plsc

N = 10000
E = 320000
D = 128
NP = 10240
NW = 32
EPT = NP
CH = EPT // 128
ROWS_PT = NP // 16
G = 128
R = 1024

f32 = jnp.float32
i32 = jnp.int32

def _deg_body(src_hbm, dst_hbm, z1_hbm, out_hbm,
              src_v, dst_v, ones_v, zb_v, oacc, iacc, sem):
    c = lax.axis_index("c")
    s = lax.axis_index("s")
    wid = s * 2 + c
    for i in range(8):
        ones_v[pl.ds(16 * i, 16)] = jnp.ones((16,), f32)
    pltpu.sync_copy(z1_hbm, zb_v)
    pltpu.sync_copy(zb_v, oacc.at[pl.ds(s * ROWS_PT, ROWS_PT)])
    pltpu.sync_copy(zb_v, iacc.at[pl.ds(s * ROWS_PT, ROWS_PT)])
    pltpu.sync_copy(src_hbm.at[wid], src_v)
    pltpu.sync_copy(dst_hbm.at[wid], dst_v)
    plsc.subcore_barrier()

    def body(j, carry):
        pltpu.sync_copy(ones_v, oacc.at[src_v.at[j]], add=True)
        pltpu.sync_copy(ones_v, iacc.at[dst_v.at[j]], add=True)
        return carry

    lax.fori_loop(0, CH, body, 0)
    plsc.subcore_barrier()
    sl = pl.ds(s * ROWS_PT, ROWS_PT)
    pltpu.sync_copy(oacc.at[sl], out_hbm.at[c, 0, sl])
    pltpu.sync_copy(iacc.at[sl], out_hbm.at[c, 1, sl])


def _hop_body(src_hbm, dst_hbm, hs_hbm, z2_hbm, out_hbm,
              src_v, dst_v, rows_v, acc, sem):
    c = lax.axis_index("c")
    s = lax.axis_index("s")
    wid = s * 2 + c
    pltpu.sync_copy(z2_hbm, rows_v)
    for i in range(ROWS_PT // 128):
        pltpu.sync_copy(rows_v, acc.at[pl.ds(s * ROWS_PT + i * 128, 128)])
    pltpu.sync_copy(src_hbm.at[wid], src_v)
    pltpu.sync_copy(dst_hbm.at[wid], dst_v)
    plsc.subcore_barrier()

    def body(j, carry):
        pltpu.async_copy(hs_hbm.at[src_v.at[j]], rows_v, sem).wait()
        pltpu.sync_copy(rows_v, acc.at[dst_v.at[j]], add=True)
        return carry

    lax.fori_loop(0, CH, body, 0)
    plsc.subcore_barrier()
    for i in range(ROWS_PT // 128):
        sl = pl.ds(s * ROWS_PT + i * 128, 128)
        pltpu.sync_copy(acc.at[sl], out_hbm.at[c, sl])


@functools.cache
def _sc_mesh():
    return plsc.VectorSubcoreMesh(
        core_axis_name="c", subcore_axis_name="s",
        num_cores=2, num_subcores=16)


@functools.cache
def _deg_kernel():
    return functools.partial(
        pl.kernel,
        out_type=jax.ShapeDtypeStruct((2, 2, NP), f32),
        mesh=_sc_mesh(),
        scratch_types=[
            pltpu.VMEM((CH, 128), i32),
            pltpu.VMEM((CH, 128), i32),
            pltpu.VMEM((128,), f32),
            pltpu.VMEM((ROWS_PT,), f32),
            pltpu.VMEM_SHARED((NP,), f32),
            pltpu.VMEM_SHARED((NP,), f32),
            pltpu.SemaphoreType.DMA,
        ])(_deg_body)


@functools.cache
def _hop_kernel():
    return functools.partial(
        pl.kernel,
        out_type=jax.ShapeDtypeStruct((2, NP, D), f32),
        mesh=_sc_mesh(),
        scratch_types=[
            pltpu.VMEM((CH, 128), i32),
            pltpu.VMEM((CH, 128), i32),
            pltpu.VMEM((128, D), f32),
            pltpu.VMEM_SHARED((NP, D), f32),
            pltpu.SemaphoreType.DMA,
        ])(_hop_body)


def _deg_sc(src_p, dst_p, z1):
    return _deg_kernel()(src_p, dst_p, z1)


def _hop_sc(src_p, dst_p, hs, z2):
    return _hop_kernel()(src_p, dst_p, hs, z2)


def _norm_body(degt_ref, feat_ref, norm2_ref, hs_ref):
    dg = degt_ref[...]
    od = dg[:, 0:1] + dg[:, 2:3]
    idg = dg[:, 1:2] + dg[:, 3:4]
    ns = 1.0 / jnp.sqrt(jnp.maximum(od, 1.0))
    nd = 1.0 / jnp.sqrt(jnp.maximum(idg, 1.0))
    norm2_ref[...] = jnp.concatenate([ns, nd], axis=1)
    hs_ref[...] = feat_ref[...] * ns


def _upd1_body(aggp_ref, norm2_ref, feat_ref, hs_ref):
    a = aggp_ref[0] + aggp_ref[1]
    n2 = norm2_ref[...]
    h = 0.5 * (a * n2[:, 1:2]) + 0.5 * feat_ref[...]
    hs_ref[...] = h * n2[:, 0:1]


def _upd2_body(aggp_ref, norm2_ref, feat_ref, t_ref, l_ref, h_ref, s_ref):
    r = pl.program_id(0)
    a = aggp_ref[0] + aggp_ref[1]
    n2 = norm2_ref[...]
    h = 0.5 * (a * n2[:, 1:2]) + 0.5 * feat_ref[...]
    h_ref[...] = h
    gid = t_ref[...] * 16 + l_ref[...]
    iota = lax.broadcasted_iota(i32, (R, G), 1)
    onehot = (iota == gid).astype(f32)
    onecol = (iota == 0).astype(f32)
    y = jnp.concatenate([h, h * h, onecol], axis=1)

    @pl.when(r == 0)
    def _init():
        s_ref[...] = jnp.zeros_like(s_ref)

    s_ref[...] += lax.dot_general(
        onehot, y, (((0,), (0,)), ((), ())),
        preferred_element_type=f32, precision=lax.Precision.HIGHEST)


def _final_body(h_ref, t_ref, l_ref, s_ref, w1t_ref, b1_ref, w2t_ref, b2_ref,
                o_ref):
    s = s_ref[...]
    cnt = jnp.maximum(s[:, 2 * G:2 * G + 1], 1.0)
    mean = s[:, 0:G] / cnt
    ex2 = s[:, G:2 * G] / cnt
    var = jnp.maximum(ex2 - mean * mean, 0.0)
    inv = 1.0 / jnp.sqrt(var + 1e-5)
    gid = t_ref[...] * 16 + l_ref[...]
    iota = lax.broadcasted_iota(i32, (R, G), 1)
    onehot = (iota == gid).astype(f32)
    hp = lax.Precision.HIGHEST
    mean_r = lax.dot_general(onehot, mean, (((1,), (0,)), ((), ())),
                             preferred_element_type=f32, precision=hp)
    inv_r = lax.dot_general(onehot, inv, (((1,), (0,)), ((), ())),
                            preferred_element_type=f32, precision=hp)
    hn = (h_ref[...] - mean_r) * inv_r
    z = lax.dot_general(hn, w1t_ref[...], (((1,), (0,)), ((), ())),
                        preferred_element_type=f32, precision=hp)
    z = jnp.maximum(z + b1_ref[...], 0.0)
    o = lax.dot_general(z, w2t_ref[...], (((1,), (0,)), ((), ())),
                        preferred_element_type=f32, precision=hp)
    o_ref[...] = jax.nn.sigmoid(o + b2_ref[...])


def _row_spec(width):
    return pl.BlockSpec((R, width), lambda r: (r, 0))


def _full_spec(shape):
    return pl.BlockSpec(shape, lambda r: tuple(0 for _ in shape))


def kernel(features, edge_index, labels, times, W1, b1, W2, b2):
    src = edge_index[0]
    dst = edge_index[1]
    npad = NP - N
    epad = NW * EPT - E
    pad_idx = (N + (jnp.arange(epad, dtype=i32) % npad)).astype(i32)
    src_p = jnp.concatenate([src, pad_idx]).reshape(NW, CH, 128)
    dst_p = jnp.concatenate([dst, pad_idx]).reshape(NW, CH, 128)
    feat_p = jnp.pad(features, ((0, npad), (0, 0)))
    t_p = jnp.pad(times, (0, npad), constant_values=8).reshape(NP, 1)
    l_p = jnp.pad(labels, (0, npad)).reshape(NP, 1)
    z1 = jnp.zeros((ROWS_PT,), f32)
    z2 = jnp.zeros((128, D), f32)

    degp = _deg_sc(src_p, dst_p, z1)
    degt = jnp.transpose(degp.reshape(4, NP))

    grid = (NP // R,)
    norm2, hs0 = pl.pallas_call(
        _norm_body,
        grid=grid,
        in_specs=[_row_spec(4), _row_spec(D)],
        out_specs=[_row_spec(2), _row_spec(D)],
        out_shape=[jax.ShapeDtypeStruct((NP, 2), f32),
                   jax.ShapeDtypeStruct((NP, D), f32)],
    )(degt, feat_p)

    aggp1 = _hop_sc(src_p, dst_p, hs0, z2)
    hs1 = pl.pallas_call(
        _upd1_body,
        grid=grid,
        in_specs=[pl.BlockSpec((2, R, D), lambda r: (0, r, 0)),
                  _row_spec(2), _row_spec(D)],
        out_specs=_row_spec(D),
        out_shape=jax.ShapeDtypeStruct((NP, D), f32),
    )(aggp1, norm2, feat_p)

    aggp2 = _hop_sc(src_p, dst_p, hs1, z2)
    h2, stats = pl.pallas_call(
        _upd2_body,
        grid=grid,
        in_specs=[pl.BlockSpec((2, R, D), lambda r: (0, r, 0)),
                  _row_spec(2), _row_spec(D), _row_spec(1), _row_spec(1)],
        out_specs=[_row_spec(D), _full_spec((G, 3 * G))],
        out_shape=[jax.ShapeDtypeStruct((NP, D), f32),
                   jax.ShapeDtypeStruct((G, 3 * G), f32)],
    )(aggp2, norm2, feat_p, t_p, l_p)

    out = pl.pallas_call(
        _final_body,
        grid=grid,
        in_specs=[_row_spec(D), _row_spec(1), _row_spec(1),
                  _full_spec((G, 3 * G)), _full_spec((D, D)),
                  _full_spec((1, D)), _full_spec((D, 32)),
                  _full_spec((1, 32))],
        out_specs=_row_spec(32),
        out_shape=jax.ShapeDtypeStruct((NP, 32), f32),
    )(h2, t_p, l_p, stats, W1.T, b1.reshape(1, D), W2.T, b2.reshape(1, 32))
    return out[:N]


# speedup vs baseline: 8.9636x; 1.1460x over previous
import functools

import jax
import jax.numpy as jnp
from jax import lax
from jax.experimental import pallas as pl
from jax.experimental.pallas import tpu as pltpu
from jax.experimental.pallas import tpu_sc as plsc

N = 10000
E = 320000
D = 128
NP = 10240
NW = 32
EPT = NP
CB = 128
CH = EPT // CB
ROWS_PT = NP // 16
G = 128
R = 1024

f32 = jnp.float32
i32 = jnp.int32

def _deg_body(src_hbm, dst_hbm, z1_hbm, out_hbm,
              src_v, dst_v, ones_v, zb_v, oacc, iacc, sem):
    c = lax.axis_index("c")
    s = lax.axis_index("s")
    wid = s * 2 + c
    for i in range(CB // 16):
        ones_v[pl.ds(16 * i, 16)] = jnp.ones((16,), f32)
    pltpu.sync_copy(z1_hbm, zb_v)
    pltpu.sync_copy(zb_v, oacc.at[pl.ds(s * ROWS_PT, ROWS_PT)])
    pltpu.sync_copy(zb_v, iacc.at[pl.ds(s * ROWS_PT, ROWS_PT)])
    pltpu.sync_copy(src_hbm.at[wid], src_v)
    pltpu.sync_copy(dst_hbm.at[wid], dst_v)
    plsc.subcore_barrier()

    def body(j, carry):
        pltpu.sync_copy(ones_v, oacc.at[src_v.at[j]], add=True)
        pltpu.sync_copy(ones_v, iacc.at[dst_v.at[j]], add=True)
        return carry

    lax.fori_loop(0, CH, body, 0)
    plsc.subcore_barrier()
    sl = pl.ds(s * ROWS_PT, ROWS_PT)
    pltpu.sync_copy(oacc.at[sl], out_hbm.at[c, 0, sl])
    pltpu.sync_copy(iacc.at[sl], out_hbm.at[c, 1, sl])


def _hop_body(src_hbm, dst_hbm, hs_hbm, z2_hbm, out_hbm,
              sr, dr, ra_v, rb_v, acc, ia, ib, ga, gb, sa, sb):
    c = lax.axis_index("c")
    s = lax.axis_index("s")
    wid = s * 2 + c
    pltpu.sync_copy(z2_hbm, ra_v)
    for i in range(ROWS_PT // CB):
        pltpu.sync_copy(ra_v, acc.at[pl.ds(s * ROWS_PT + i * CB, CB)])
    plsc.subcore_barrier()

    pltpu.async_copy(src_hbm.at[wid, pl.ds(0, 2)], sr.at[0], ia)
    pltpu.async_copy(dst_hbm.at[wid, pl.ds(0, 2)], dr.at[0], ia)
    pltpu.async_copy(src_hbm.at[wid, pl.ds(2, 2)], sr.at[1], ib)
    pltpu.async_copy(dst_hbm.at[wid, pl.ds(2, 2)], dr.at[1], ib)
    pltpu.make_async_copy(src_hbm.at[wid, pl.ds(0, 2)], sr.at[0], ia).wait()
    pltpu.make_async_copy(dst_hbm.at[wid, pl.ds(0, 2)], dr.at[0], ia).wait()
    pltpu.async_copy(hs_hbm.at[sr.at[0, 0]], ra_v, ga)
    pltpu.async_copy(hs_hbm.at[sr.at[0, 1]], rb_v, gb)

    def halfiter(j0, slot, oslot, isem_o, isem_s):
        pltpu.make_async_copy(hs_hbm.at[sr.at[slot, 0]], ra_v, ga).wait()
        pltpu.async_copy(ra_v, acc.at[dr.at[slot, 0]], sa, add=True)
        pltpu.make_async_copy(hs_hbm.at[sr.at[slot, 1]], rb_v, gb).wait()
        pltpu.async_copy(rb_v, acc.at[dr.at[slot, 1]], sb, add=True)

        @pl.when(j0 + 2 < CH)
        def _pf():
            pltpu.make_async_copy(
                src_hbm.at[wid, pl.ds(0, 2)], sr.at[oslot], isem_o).wait()
            pltpu.make_async_copy(
                dst_hbm.at[wid, pl.ds(0, 2)], dr.at[oslot], isem_o).wait()
            pltpu.make_async_copy(ra_v, acc.at[dr.at[slot, 0]], sa).wait()
            pltpu.async_copy(hs_hbm.at[sr.at[oslot, 0]], ra_v, ga)
            pltpu.make_async_copy(rb_v, acc.at[dr.at[slot, 1]], sb).wait()
            pltpu.async_copy(hs_hbm.at[sr.at[oslot, 1]], rb_v, gb)

            @pl.when(j0 + 4 < CH)
            def _refill():
                pltpu.async_copy(
                    src_hbm.at[wid, pl.ds(j0 + 4, 2)], sr.at[slot], isem_s)
                pltpu.async_copy(
                    dst_hbm.at[wid, pl.ds(j0 + 4, 2)], dr.at[slot], isem_s)

    def body(k, carry):
        j0 = 4 * k
        halfiter(j0, 0, 1, ib, ia)
        halfiter(j0 + 2, 1, 0, ia, ib)
        return carry

    lax.fori_loop(0, CH // 4, body, 0)
    pltpu.make_async_copy(ra_v, acc.at[dr.at[1, 0]], sa).wait()
    pltpu.make_async_copy(rb_v, acc.at[dr.at[1, 1]], sb).wait()
    plsc.subcore_barrier()
    for i in range(ROWS_PT // 128):
        sl = pl.ds(s * ROWS_PT + i * 128, 128)
        pltpu.sync_copy(acc.at[sl], out_hbm.at[c, sl])


@functools.cache
def _sc_mesh():
    return plsc.VectorSubcoreMesh(
        core_axis_name="c", subcore_axis_name="s",
        num_cores=2, num_subcores=16)


@functools.cache
def _deg_kernel():
    return functools.partial(
        pl.kernel,
        out_type=jax.ShapeDtypeStruct((2, 2, NP), f32),
        mesh=_sc_mesh(),
        scratch_types=[
            pltpu.VMEM((CH, CB), i32),
            pltpu.VMEM((CH, CB), i32),
            pltpu.VMEM((CB,), f32),
            pltpu.VMEM((ROWS_PT,), f32),
            pltpu.VMEM_SHARED((NP,), f32),
            pltpu.VMEM_SHARED((NP,), f32),
            pltpu.SemaphoreType.DMA,
        ])(_deg_body)


@functools.cache
def _hop_kernel():
    return functools.partial(
        pl.kernel,
        out_type=jax.ShapeDtypeStruct((2, NP, D), f32),
        mesh=_sc_mesh(),
        scratch_types=[
            pltpu.VMEM((2, 2, CB), i32),
            pltpu.VMEM((2, 2, CB), i32),
            pltpu.VMEM((CB, D), f32),
            pltpu.VMEM((CB, D), f32),
            pltpu.VMEM_SHARED((NP, D), f32),
            pltpu.SemaphoreType.DMA,
            pltpu.SemaphoreType.DMA,
            pltpu.SemaphoreType.DMA,
            pltpu.SemaphoreType.DMA,
            pltpu.SemaphoreType.DMA,
            pltpu.SemaphoreType.DMA,
        ])(_hop_body)


def _deg_sc(src_p, dst_p, z1):
    return _deg_kernel()(src_p, dst_p, z1)


def _hop_sc(src_p, dst_p, hs, z2):
    return _hop_kernel()(src_p, dst_p, hs, z2)


def _norm_body(degt_ref, feat_ref, norm2_ref, hs_ref):
    dg = degt_ref[...]
    od = dg[:, 0:1] + dg[:, 2:3]
    idg = dg[:, 1:2] + dg[:, 3:4]
    ns = 1.0 / jnp.sqrt(jnp.maximum(od, 1.0))
    nd = 1.0 / jnp.sqrt(jnp.maximum(idg, 1.0))
    norm2_ref[...] = jnp.concatenate([ns, nd], axis=1)
    hs_ref[...] = feat_ref[...] * ns


def _upd1_body(aggp_ref, norm2_ref, feat_ref, hs_ref):
    a = aggp_ref[0] + aggp_ref[1]
    n2 = norm2_ref[...]
    h = 0.5 * (a * n2[:, 1:2]) + 0.5 * feat_ref[...]
    hs_ref[...] = h * n2[:, 0:1]


def _upd2_body(aggp_ref, norm2_ref, feat_ref, t_ref, l_ref, h_ref, s_ref):
    r = pl.program_id(0)
    a = aggp_ref[0] + aggp_ref[1]
    n2 = norm2_ref[...]
    h = 0.5 * (a * n2[:, 1:2]) + 0.5 * feat_ref[...]
    h_ref[...] = h
    gid = t_ref[...] * 16 + l_ref[...]
    iota = lax.broadcasted_iota(i32, (R, G), 1)
    onehot = (iota == gid).astype(f32)
    onecol = (iota == 0).astype(f32)
    y = jnp.concatenate([h, h * h, onecol], axis=1)

    @pl.when(r == 0)
    def _init():
        s_ref[...] = jnp.zeros_like(s_ref)

    s_ref[...] += lax.dot_general(
        onehot, y, (((0,), (0,)), ((), ())),
        preferred_element_type=f32, precision=lax.Precision.HIGHEST)


def _final_body(h_ref, t_ref, l_ref, s_ref, w1t_ref, b1_ref, w2t_ref, b2_ref,
                o_ref):
    s = s_ref[...]
    cnt = jnp.maximum(s[:, 2 * G:2 * G + 1], 1.0)
    mean = s[:, 0:G] / cnt
    ex2 = s[:, G:2 * G] / cnt
    var = jnp.maximum(ex2 - mean * mean, 0.0)
    inv = 1.0 / jnp.sqrt(var + 1e-5)
    gid = t_ref[...] * 16 + l_ref[...]
    iota = lax.broadcasted_iota(i32, (R, G), 1)
    onehot = (iota == gid).astype(f32)
    hp = lax.Precision.HIGHEST
    mean_r = lax.dot_general(onehot, mean, (((1,), (0,)), ((), ())),
                             preferred_element_type=f32, precision=hp)
    inv_r = lax.dot_general(onehot, inv, (((1,), (0,)), ((), ())),
                            preferred_element_type=f32, precision=hp)
    hn = (h_ref[...] - mean_r) * inv_r
    z = lax.dot_general(hn, w1t_ref[...], (((1,), (0,)), ((), ())),
                        preferred_element_type=f32, precision=hp)
    z = jnp.maximum(z + b1_ref[...], 0.0)
    o = lax.dot_general(z, w2t_ref[...], (((1,), (0,)), ((), ())),
                        preferred_element_type=f32, precision=hp)
    o_ref[...] = jax.nn.sigmoid(o + b2_ref[...])


def _row_spec(width):
    return pl.BlockSpec((R, width), lambda r: (r, 0))


def _full_spec(shape):
    return pl.BlockSpec(shape, lambda r: tuple(0 for _ in shape))


def kernel(features, edge_index, labels, times, W1, b1, W2, b2):
    src = edge_index[0]
    dst = edge_index[1]
    npad = NP - N
    epad = NW * EPT - E
    pad_idx = (N + (jnp.arange(epad, dtype=i32) % npad)).astype(i32)
    src_p = jnp.concatenate([src, pad_idx]).reshape(NW, CH, CB)
    dst_p = jnp.concatenate([dst, pad_idx]).reshape(NW, CH, CB)
    feat_p = jnp.pad(features, ((0, npad), (0, 0)))
    t_p = jnp.pad(times, (0, npad), constant_values=8).reshape(NP, 1)
    l_p = jnp.pad(labels, (0, npad)).reshape(NP, 1)
    z1 = jnp.zeros((ROWS_PT,), f32)
    z2 = jnp.zeros((CB, D), f32)

    degp = _deg_sc(src_p, dst_p, z1)
    degt = jnp.transpose(degp.reshape(4, NP))

    grid = (NP // R,)
    norm2, hs0 = pl.pallas_call(
        _norm_body,
        grid=grid,
        in_specs=[_row_spec(4), _row_spec(D)],
        out_specs=[_row_spec(2), _row_spec(D)],
        out_shape=[jax.ShapeDtypeStruct((NP, 2), f32),
                   jax.ShapeDtypeStruct((NP, D), f32)],
    )(degt, feat_p)

    aggp1 = _hop_sc(src_p, dst_p, hs0, z2)
    hs1 = pl.pallas_call(
        _upd1_body,
        grid=grid,
        in_specs=[pl.BlockSpec((2, R, D), lambda r: (0, r, 0)),
                  _row_spec(2), _row_spec(D)],
        out_specs=_row_spec(D),
        out_shape=jax.ShapeDtypeStruct((NP, D), f32),
    )(aggp1, norm2, feat_p)

    aggp2 = _hop_sc(src_p, dst_p, hs1, z2)
    h2, stats = pl.pallas_call(
        _upd2_body,
        grid=grid,
        in_specs=[pl.BlockSpec((2, R, D), lambda r: (0, r, 0)),
                  _row_spec(2), _row_spec(D), _row_spec(1), _row_spec(1)],
        out_specs=[_row_spec(D), _full_spec((G, 3 * G))],
        out_shape=[jax.ShapeDtypeStruct((NP, D), f32),
                   jax.ShapeDtypeStruct((G, 3 * G), f32)],
    )(aggp2, norm2, feat_p, t_p, l_p)

    out = pl.pallas_call(
        _final_body,
        grid=grid,
        in_specs=[_row_spec(D), _row_spec(1), _row_spec(1),
                  _full_spec((G, 3 * G)), _full_spec((D, D)),
                  _full_spec((1, D)), _full_spec((D, 32)),
                  _full_spec((1, 32))],
        out_specs=_row_spec(32),
        out_shape=jax.ShapeDtypeStruct((NP, 32), f32),
    )(h2, t_p, l_p, stats, W1.T, b1.reshape(1, D), W2.T, b2.reshape(1, 32))
    return out[:N]


# speedup vs baseline: 9.2908x; 1.0365x over previous
import functools

import jax
import jax.numpy as jnp
from jax import lax
from jax.experimental import pallas as pl
from jax.experimental.pallas import tpu as pltpu
from jax.experimental.pallas import tpu_sc as plsc

N = 10000
E = 320000
D = 128
NP = 10240
NW = 32
EPT = NP
CB = 128
CH = EPT // CB
ROWS_PT = NP // 16
G = 128
R = 1024

f32 = jnp.float32
i32 = jnp.int32

def _deg_body(ei2_hbm, z1_hbm, out_hbm, src_v, dst_v, ones_v, zb_v, acc, sem):
    c = lax.axis_index("c")
    s = lax.axis_index("s")
    wid = s * 2 + c
    for i in range(CB // 16):
        ones_v[pl.ds(16 * i, 16)] = jnp.ones((16,), f32)
    sl2 = pl.ds(s * 2 * ROWS_PT, 2 * ROWS_PT)
    pltpu.sync_copy(z1_hbm, zb_v)
    pltpu.sync_copy(zb_v, acc.at[sl2])
    pltpu.sync_copy(ei2_hbm.at[0, wid], src_v)
    pltpu.sync_copy(ei2_hbm.at[1, wid], dst_v)
    plsc.subcore_barrier()

    def body(j, carry):
        pltpu.sync_copy(ones_v, acc.at[src_v.at[j]], add=True)
        pltpu.sync_copy(ones_v, acc.at[dst_v.at[j]], add=True)
        return carry

    lax.fori_loop(0, CH, body, 0)
    plsc.subcore_barrier()
    pltpu.sync_copy(acc.at[sl2], out_hbm.at[c, sl2])


def _hop_body(ei_hbm, hs_hbm, z2_hbm, out_hbm,
              sr, dr, ra_v, rb_v, acc, ia, ib, ga, gb, sa, sb):
    c = lax.axis_index("c")
    s = lax.axis_index("s")
    wid = s * 2 + c
    pltpu.sync_copy(z2_hbm, ra_v)
    for i in range(ROWS_PT // CB):
        pltpu.sync_copy(ra_v, acc.at[pl.ds(s * ROWS_PT + i * CB, CB)])
    plsc.subcore_barrier()

    pltpu.async_copy(ei_hbm.at[0, wid, pl.ds(0, 2)], sr.at[0], ia)
    pltpu.async_copy(ei_hbm.at[1, wid, pl.ds(0, 2)], dr.at[0], ia)
    pltpu.async_copy(ei_hbm.at[0, wid, pl.ds(2, 2)], sr.at[1], ib)
    pltpu.async_copy(ei_hbm.at[1, wid, pl.ds(2, 2)], dr.at[1], ib)
    pltpu.make_async_copy(ei_hbm.at[0, wid, pl.ds(0, 2)], sr.at[0], ia).wait()
    pltpu.make_async_copy(ei_hbm.at[1, wid, pl.ds(0, 2)], dr.at[0], ia).wait()
    pltpu.async_copy(hs_hbm.at[sr.at[0, 0]], ra_v, ga)
    pltpu.async_copy(hs_hbm.at[sr.at[0, 1]], rb_v, gb)

    def halfiter(j0, slot, oslot, isem_o, isem_s):
        pltpu.make_async_copy(hs_hbm.at[sr.at[slot, 0]], ra_v, ga).wait()
        pltpu.async_copy(ra_v, acc.at[dr.at[slot, 0]], sa, add=True)
        pltpu.make_async_copy(hs_hbm.at[sr.at[slot, 1]], rb_v, gb).wait()
        pltpu.async_copy(rb_v, acc.at[dr.at[slot, 1]], sb, add=True)

        @pl.when(j0 + 2 < CH)
        def _pf():
            pltpu.make_async_copy(
                ei_hbm.at[0, wid, pl.ds(0, 2)], sr.at[oslot], isem_o).wait()
            pltpu.make_async_copy(
                ei_hbm.at[1, wid, pl.ds(0, 2)], dr.at[oslot], isem_o).wait()
            pltpu.make_async_copy(ra_v, acc.at[dr.at[slot, 0]], sa).wait()
            pltpu.async_copy(hs_hbm.at[sr.at[oslot, 0]], ra_v, ga)
            pltpu.make_async_copy(rb_v, acc.at[dr.at[slot, 1]], sb).wait()
            pltpu.async_copy(hs_hbm.at[sr.at[oslot, 1]], rb_v, gb)

            @pl.when(j0 + 4 < CH)
            def _refill():
                pltpu.async_copy(
                    ei_hbm.at[0, wid, pl.ds(j0 + 4, 2)], sr.at[slot], isem_s)
                pltpu.async_copy(
                    ei_hbm.at[1, wid, pl.ds(j0 + 4, 2)], dr.at[slot], isem_s)

    def body(k, carry):
        j0 = 4 * k
        halfiter(j0, 0, 1, ib, ia)
        halfiter(j0 + 2, 1, 0, ia, ib)
        return carry

    lax.fori_loop(0, CH // 4, body, 0)
    pltpu.make_async_copy(ra_v, acc.at[dr.at[1, 0]], sa).wait()
    pltpu.make_async_copy(rb_v, acc.at[dr.at[1, 1]], sb).wait()
    plsc.subcore_barrier()
    for i in range(ROWS_PT // 128):
        sl = pl.ds(s * ROWS_PT + i * 128, 128)
        pltpu.sync_copy(acc.at[sl], out_hbm.at[c, sl])


@functools.cache
def _sc_mesh():
    return plsc.VectorSubcoreMesh(
        core_axis_name="c", subcore_axis_name="s",
        num_cores=2, num_subcores=16)


@functools.cache
def _deg_kernel():
    return functools.partial(
        pl.kernel,
        out_type=jax.ShapeDtypeStruct((2, 2 * NP), f32),
        mesh=_sc_mesh(),
        scratch_types=[
            pltpu.VMEM((CH, CB), i32),
            pltpu.VMEM((CH, CB), i32),
            pltpu.VMEM((CB,), f32),
            pltpu.VMEM((2 * ROWS_PT,), f32),
            pltpu.VMEM_SHARED((2 * NP,), f32),
            pltpu.SemaphoreType.DMA,
        ])(_deg_body)


@functools.cache
def _hop_kernel():
    return functools.partial(
        pl.kernel,
        out_type=jax.ShapeDtypeStruct((2, NP, D), f32),
        mesh=_sc_mesh(),
        scratch_types=[
            pltpu.VMEM((2, 2, CB), i32),
            pltpu.VMEM((2, 2, CB), i32),
            pltpu.VMEM((CB, D), f32),
            pltpu.VMEM((CB, D), f32),
            pltpu.VMEM_SHARED((NP, D), f32),
            pltpu.SemaphoreType.DMA,
            pltpu.SemaphoreType.DMA,
            pltpu.SemaphoreType.DMA,
            pltpu.SemaphoreType.DMA,
            pltpu.SemaphoreType.DMA,
            pltpu.SemaphoreType.DMA,
        ])(_hop_body)


def _deg_sc(ei2, z1):
    return _deg_kernel()(ei2, z1)


def _hop_sc(ei_p, hs, z2):
    return _hop_kernel()(ei_p, hs, z2)


def _norm_body(degp_ref, feat_ref, norm2_ref, hs_ref):
    od = degp_ref[0, :, 0:1] + degp_ref[1, :, 0:1]
    idg = degp_ref[0, :, 1:2] + degp_ref[1, :, 1:2]
    ns = 1.0 / jnp.sqrt(jnp.maximum(od, 1.0))
    nd = 1.0 / jnp.sqrt(jnp.maximum(idg, 1.0))
    norm2_ref[...] = jnp.concatenate([ns, nd], axis=1)
    hs_ref[...] = feat_ref[...] * ns


def _upd1_body(aggp_ref, norm2_ref, feat_ref, hs_ref):
    a = aggp_ref[0] + aggp_ref[1]
    n2 = norm2_ref[...]
    h = 0.5 * (a * n2[:, 1:2]) + 0.5 * feat_ref[...]
    hs_ref[...] = h * n2[:, 0:1]


def _upd2_body(aggp_ref, norm2_ref, feat_ref, t_ref, l_ref, h_ref, s_ref):
    r = pl.program_id(0)
    a = aggp_ref[0] + aggp_ref[1]
    n2 = norm2_ref[...]
    h = 0.5 * (a * n2[:, 1:2]) + 0.5 * feat_ref[...]
    h_ref[...] = h
    gid = t_ref[...] * 16 + l_ref[...]
    iota = lax.broadcasted_iota(i32, (R, G), 1)
    onehot = (iota == gid).astype(f32)
    onecol = (iota == 0).astype(f32)
    rowid = lax.broadcasted_iota(i32, (R, 1), 0) + r * R
    h_m = jnp.where(rowid < N, h, 0.0)
    y = jnp.concatenate([h_m, h_m * h_m, onecol], axis=1)

    @pl.when(r == 0)
    def _init():
        s_ref[...] = jnp.zeros_like(s_ref)

    s_ref[...] += lax.dot_general(
        onehot, y, (((0,), (0,)), ((), ())), preferred_element_type=f32)


def _final_body(h_ref, t_ref, l_ref, s_ref, w1t_ref, b1_ref, w2t_ref, b2_ref,
                o_ref):
    s = s_ref[...]
    cnt = jnp.maximum(s[:, 2 * G:2 * G + 1], 1.0)
    mean = s[:, 0:G] / cnt
    ex2 = s[:, G:2 * G] / cnt
    var = jnp.maximum(ex2 - mean * mean, 0.0)
    inv = 1.0 / jnp.sqrt(var + 1e-5)
    gid = t_ref[...] * 16 + l_ref[...]
    iota = lax.broadcasted_iota(i32, (R, G), 1)
    onehot = (iota == gid).astype(f32)
    mean_r = lax.dot_general(onehot, mean, (((1,), (0,)), ((), ())),
                             preferred_element_type=f32)
    inv_r = lax.dot_general(onehot, inv, (((1,), (0,)), ((), ())),
                            preferred_element_type=f32)
    hn = (h_ref[...] - mean_r) * inv_r
    z = lax.dot_general(hn, w1t_ref[...], (((1,), (0,)), ((), ())),
                        preferred_element_type=f32)
    z = jnp.maximum(z + b1_ref[...], 0.0)
    o = lax.dot_general(z, w2t_ref[...], (((1,), (0,)), ((), ())),
                        preferred_element_type=f32)
    o_ref[...] = jax.nn.sigmoid(o + b2_ref[...])


def _row_spec(width):
    return pl.BlockSpec((R, width), lambda r: (r, 0))


def _full_spec(shape):
    return pl.BlockSpec(shape, lambda r: tuple(0 for _ in shape))


def kernel(features, edge_index, labels, times, W1, b1, W2, b2):
    npad = NP - N
    epad = NW * EPT - E
    pad_idx = (N + (jnp.arange(epad, dtype=i32) % npad)).astype(i32)
    ei_p = jnp.concatenate(
        [edge_index, jnp.broadcast_to(pad_idx, (2, epad))],
        axis=1).reshape(2, NW, CH, CB)
    t_p = jnp.pad(times, (0, npad), constant_values=8).reshape(NP, 1)
    l_p = jnp.pad(labels, (0, npad)).reshape(NP, 1)
    z1 = jnp.zeros((2 * ROWS_PT,), f32)
    z2 = jnp.zeros((CB, D), f32)

    ei2 = ei_p * 2 + jnp.arange(2, dtype=i32).reshape(2, 1, 1, 1)
    degp2 = _deg_sc(ei2, z1).reshape(2, NP, 2)

    grid = (NP // R,)
    norm2, hs0 = pl.pallas_call(
        _norm_body,
        grid=grid,
        in_specs=[pl.BlockSpec((2, R, 2), lambda r: (0, r, 0)), _row_spec(D)],
        out_specs=[_row_spec(2), _row_spec(D)],
        out_shape=[jax.ShapeDtypeStruct((NP, 2), f32),
                   jax.ShapeDtypeStruct((NP, D), f32)],
    )(degp2, features)

    aggp1 = _hop_sc(ei_p, hs0, z2)
    hs1 = pl.pallas_call(
        _upd1_body,
        grid=grid,
        in_specs=[pl.BlockSpec((2, R, D), lambda r: (0, r, 0)),
                  _row_spec(2), _row_spec(D)],
        out_specs=_row_spec(D),
        out_shape=jax.ShapeDtypeStruct((NP, D), f32),
    )(aggp1, norm2, features)

    aggp2 = _hop_sc(ei_p, hs1, z2)
    h2, stats = pl.pallas_call(
        _upd2_body,
        grid=grid,
        in_specs=[pl.BlockSpec((2, R, D), lambda r: (0, r, 0)),
                  _row_spec(2), _row_spec(D), _row_spec(1), _row_spec(1)],
        out_specs=[_row_spec(D), _full_spec((G, 3 * G))],
        out_shape=[jax.ShapeDtypeStruct((NP, D), f32),
                   jax.ShapeDtypeStruct((G, 3 * G), f32)],
    )(aggp2, norm2, features, t_p, l_p)

    out = pl.pallas_call(
        _final_body,
        grid=grid,
        in_specs=[_row_spec(D), _row_spec(1), _row_spec(1),
                  _full_spec((G, 3 * G)), _full_spec((D, D)),
                  _full_spec((1, D)), _full_spec((D, 32)),
                  _full_spec((1, 32))],
        out_specs=_row_spec(32),
        out_shape=jax.ShapeDtypeStruct((NP, 32), f32),
    )(h2, t_p, l_p, stats, W1.T, b1.reshape(1, D), W2.T, b2.reshape(1, 32))
    return out[:N]
